# Initial kernel scaffold; baseline (speedup 1.0000x reference)
import jax
import jax.numpy as jnp
from jax.experimental import pallas as pl


def kernel(t, x_nodes, edge_index, edge_weight, W1, b1, gamma1, beta1, W2, b2):
    raise NotImplementedError("write your pallas kernel here")


# speedup vs baseline: 9.7760x; 9.7760x over previous
import functools

import jax
import jax.numpy as jnp
from jax import lax
from jax.experimental import pallas as pl
from jax.experimental.pallas import tpu as pltpu
from jax.experimental.pallas import tpu_sc as plsc

N = 10000
D = 128
E = 320000
EPS = 1e-5

NC = 2
NS = 16
NW = NC * NS
EPT = E // NW
BLK = 80
NB = EPT // BLK
NPAD = 10240
DWPT = NPAD // NS
RPT = NPAD // NS
ZROWS = 128

_mesh = plsc.VectorSubcoreMesh(core_axis_name="c", subcore_axis_name="s")


@functools.partial(
    pl.kernel,
    out_type=jax.ShapeDtypeStruct((NC, NPAD), jnp.float32),
    mesh=_mesh,
    scratch_types=[
        pltpu.VMEM((DWPT,), jnp.float32),
        pltpu.VMEM((BLK,), jnp.int32),
        pltpu.VMEM((BLK,), jnp.float32),
        pltpu.VMEM_SHARED((NPAD,), jnp.float32),
    ],
)
def _deg_call(dst_hbm, ew_hbm, out_hbm, zb, dstv, ewv, accd):
    c = lax.axis_index("c")
    s = lax.axis_index("s")

    @pl.loop(0, DWPT // 16)
    def _zero(i):
        zb[pl.ds(i * 16, 16)] = jnp.zeros((16,), jnp.float32)

    pltpu.sync_copy(zb, accd.at[pl.ds(s * DWPT, DWPT)])
    plsc.subcore_barrier()

    wid = c * NS + s

    @pl.loop(0, NB)
    def _blocks(i):
        off = wid * EPT + i * BLK
        pltpu.sync_copy(dst_hbm.at[pl.ds(off, BLK)], dstv)
        pltpu.sync_copy(ew_hbm.at[pl.ds(off, BLK)], ewv)
        pltpu.sync_copy(ewv, accd.at[dstv], add=True)

    plsc.subcore_barrier()
    pltpu.sync_copy(accd.at[pl.ds(s * DWPT, DWPT)],
                    out_hbm.at[c, pl.ds(s * DWPT, DWPT)])


@functools.partial(
    pl.kernel,
    out_type=jax.ShapeDtypeStruct((NC, NPAD, D), jnp.float32),
    mesh=_mesh,
    scratch_types=[
        pltpu.VMEM((ZROWS, D), jnp.float32),
        pltpu.VMEM((BLK,), jnp.int32),
        pltpu.VMEM((BLK,), jnp.int32),
        pltpu.VMEM((BLK,), jnp.float32),
        pltpu.VMEM((BLK, D), jnp.float32),
        pltpu.VMEM_SHARED((NPAD, D), jnp.float32),
        pltpu.SemaphoreType.DMA,
    ],
)
def _edge_call(g_hbm, src_hbm, dst_hbm, ew_hbm, out_hbm,
               zb, srcv, dstv, ewv, rows, acc, sem):
    c = lax.axis_index("c")
    s = lax.axis_index("s")

    @pl.loop(0, ZROWS)
    def _zero(r):
        for j in range(D // 16):
            zb[r, pl.ds(j * 16, 16)] = jnp.zeros((16,), jnp.float32)

    for k in range(RPT // ZROWS):
        pltpu.sync_copy(zb, acc.at[pl.ds(s * RPT + k * ZROWS, ZROWS)])
    plsc.subcore_barrier()

    wid = c * NS + s

    @pl.loop(0, NB)
    def _blocks(i):
        off = wid * EPT + i * BLK
        pltpu.sync_copy(src_hbm.at[pl.ds(off, BLK)], srcv)
        pltpu.sync_copy(dst_hbm.at[pl.ds(off, BLK)], dstv)
        pltpu.sync_copy(ew_hbm.at[pl.ds(off, BLK)], ewv)
        pltpu.async_copy(g_hbm.at[srcv], rows, sem).wait()

        @pl.loop(0, BLK // 16)
        def _scale(b16):
            wv = ewv[pl.ds(b16 * 16, 16)]
            for k in range(16):
                spl = wv[k]
                b = b16 * 16 + k
                for j in range(D // 16):
                    sl = pl.ds(j * 16, 16)
                    rows[b, sl] = rows[b, sl] * spl

        pltpu.sync_copy(rows, acc.at[dstv], add=True)

    plsc.subcore_barrier()
    for k in range(RPT // ZROWS):
        r0 = s * RPT + k * ZROWS
        pltpu.sync_copy(acc.at[pl.ds(r0, ZROWS)], out_hbm.at[c, pl.ds(r0, ZROWS)])


def _dense1_body(x_ref, w_ref, deg_ref, h1_ref, g1_ref):
    h1 = jnp.dot(x_ref[...], w_ref[...], preferred_element_type=jnp.float32)
    dinv = lax.rsqrt(deg_ref[...] + 1.0)
    h1_ref[...] = h1
    g1_ref[...] = h1 * dinv


def _dense1(x, W1, deg_col):
    return pl.pallas_call(
        _dense1_body,
        out_shape=(jax.ShapeDtypeStruct((N, D), jnp.float32),
                   jax.ShapeDtypeStruct((N, D), jnp.float32)),
    )(x, W1, deg_col)


def _dense2_body(s0_ref, s1_ref, h1_ref, deg_ref, gamma_ref, beta_ref, b1_ref,
                 w2_ref, h2_ref, g2_ref):
    dinv = lax.rsqrt(deg_ref[...] + 1.0)
    h = dinv * (s0_ref[...] + s1_ref[...]) + (dinv * dinv) * h1_ref[...] \
        + b1_ref[...]
    mean = jnp.mean(h, axis=0, keepdims=True)
    hm = h - mean
    var = jnp.mean(hm * hm, axis=0, keepdims=True)
    hn = hm * lax.rsqrt(var + EPS) * gamma_ref[...] + beta_ref[...]
    r = jnp.maximum(hn, 0.0)
    h2 = jnp.dot(r, w2_ref[...], preferred_element_type=jnp.float32)
    h2_ref[...] = h2
    g2_ref[...] = h2 * dinv


def _dense2(s0, s1, h1, deg_col, gamma, beta, b1, W2):
    return pl.pallas_call(
        _dense2_body,
        out_shape=(jax.ShapeDtypeStruct((N, D), jnp.float32),
                   jax.ShapeDtypeStruct((N, D), jnp.float32)),
    )(s0, s1, h1, deg_col, gamma, beta, b1, W2)


def _dense3_body(s0_ref, s1_ref, h2_ref, deg_ref, b2_ref, out_ref):
    dinv = lax.rsqrt(deg_ref[...] + 1.0)
    out_ref[...] = dinv * (s0_ref[...] + s1_ref[...]) \
        + (dinv * dinv) * h2_ref[...] + b2_ref[...]


def _dense3(s0, s1, h2, deg_col, b2):
    return pl.pallas_call(
        _dense3_body,
        out_shape=jax.ShapeDtypeStruct((N, D), jnp.float32),
    )(s0, s1, h2, deg_col, b2)


def kernel(t, x_nodes, edge_index, edge_weight, W1, b1, gamma1, beta1, W2, b2):
    src = edge_index[0]
    dst = edge_index[1]

    degp = _deg_call(dst, edge_weight)
    deg_col = (degp[0, :N] + degp[1, :N]).reshape(N, 1)

    h1, g1 = _dense1(x_nodes, W1, deg_col)
    s1 = _edge_call(g1, src, dst, edge_weight)
    h2, g2 = _dense2(s1[0, :N], s1[1, :N], h1, deg_col,
                     gamma1.reshape(1, D), beta1.reshape(1, D),
                     b1.reshape(1, D), W2)
    s2 = _edge_call(g2, src, dst, edge_weight)
    dz = _dense3(s2[0, :N], s2[1, :N], h2, deg_col, b2.reshape(1, D))
    return dz


# speedup vs baseline: 25.7795x; 2.6370x over previous
import functools

import jax
import jax.numpy as jnp
from jax import lax
from jax.experimental import pallas as pl
from jax.experimental.pallas import tpu as pltpu
from jax.experimental.pallas import tpu_sc as plsc

N = 10000
D = 128
E = 320000
EPS = 1e-5

NC = 2
NS = 16
NW = NC * NS
EPT = E // NW
BLK = 80
NB = EPT // BLK
NPAD = 10240
DWPT = NPAD // NS
RPT = NPAD // NS
ZROWS = 128

_mesh = plsc.VectorSubcoreMesh(core_axis_name="c", subcore_axis_name="s")


@functools.partial(
    pl.kernel,
    out_type=jax.ShapeDtypeStruct((NC, NPAD), jnp.float32),
    mesh=_mesh,
    scratch_types=[
        pltpu.VMEM((DWPT,), jnp.float32),
        pltpu.VMEM((NB, BLK), jnp.int32),
        pltpu.VMEM((NB, BLK), jnp.float32),
        pltpu.VMEM_SHARED((NPAD,), jnp.float32),
    ],
)
def _deg_call(dst_hbm, ew_hbm, out_hbm, zb, dsti, ewi, accd):
    c = lax.axis_index("c")
    s = lax.axis_index("s")
    wid = c * NS + s

    pltpu.sync_copy(dst_hbm.at[wid], dsti)
    pltpu.sync_copy(ew_hbm.at[wid], ewi)

    @pl.loop(0, DWPT // 16)
    def _zero(i):
        zb[pl.ds(i * 16, 16)] = jnp.zeros((16,), jnp.float32)

    pltpu.sync_copy(zb, accd.at[pl.ds(s * DWPT, DWPT)])
    plsc.subcore_barrier()

    @pl.loop(0, NB)
    def _blocks(i):
        pltpu.sync_copy(ewi.at[i], accd.at[dsti.at[i]], add=True)

    plsc.subcore_barrier()
    pltpu.sync_copy(accd.at[pl.ds(s * DWPT, DWPT)],
                    out_hbm.at[c, pl.ds(s * DWPT, DWPT)])


@functools.partial(
    pl.kernel,
    out_type=jax.ShapeDtypeStruct((NC, NPAD, D), jnp.float32),
    mesh=_mesh,
    scratch_types=[
        pltpu.VMEM((EPT,), jnp.int32),
        pltpu.VMEM((EPT,), jnp.int32),
        pltpu.VMEM((BLK,), jnp.int32),
        pltpu.VMEM((BLK,), jnp.int32),
        pltpu.VMEM((BLK,), jnp.float32),
        pltpu.VMEM((BLK,), jnp.float32),
        pltpu.VMEM((BLK, D), jnp.float32),
        pltpu.VMEM((BLK, D), jnp.float32),
        pltpu.VMEM_SHARED((NPAD, D), jnp.float32),
        pltpu.SemaphoreType.DMA,
        pltpu.SemaphoreType.DMA,
    ],
)
def _edge_call(g_hbm, src_hbm, dst_hbm, ew_hbm, out_hbm,
               srcf, dstf, dstv0, dstv1, ewv0, ewv1, rows0, rows1,
               acc, sem0, sem1):
    c = lax.axis_index("c")
    s = lax.axis_index("s")
    wid = c * NS + s

    pltpu.sync_copy(src_hbm.at[pl.ds(wid * EPT, EPT)], srcf)
    pltpu.sync_copy(dst_hbm.at[pl.ds(wid * EPT, EPT)], dstf)

    @pl.loop(0, BLK)
    def _zero(r):
        for j in range(D // 16):
            rows0[r, pl.ds(j * 16, 16)] = jnp.zeros((16,), jnp.float32)

    for k in range(RPT // BLK):
        pltpu.sync_copy(rows0, acc.at[pl.ds(s * RPT + k * BLK, BLK)])
    plsc.subcore_barrier()

    def _start_blk(i, dstv, ewv, rows, sem):
        for k in range(BLK // 16):
            dstv[pl.ds(k * 16, 16)] = dstf[pl.ds(i * BLK + k * 16, 16)]
        pltpu.async_copy(ew_hbm.at[pl.ds(wid * EPT + i * BLK, BLK)], ewv, sem)
        pltpu.async_copy(g_hbm.at[srcf.at[pl.ds(i * BLK, BLK)]], rows, sem)

    def _wait_blk(i, ewv, rows, sem):
        pltpu.make_async_copy(
            ew_hbm.at[pl.ds(wid * EPT + i * BLK, BLK)], ewv, sem).wait()
        pltpu.make_async_copy(
            g_hbm.at[srcf.at[pl.ds(i * BLK, BLK)]], rows, sem).wait()

    def _scale(ewv, buf):
        @pl.loop(0, BLK // 16)
        def _s(b16):
            wv = ewv[pl.ds(b16 * 16, 16)]
            for k in range(16):
                spl = wv[k]
                b = b16 * 16 + k
                for j in range(D // 16):
                    sl = pl.ds(j * 16, 16)
                    buf[b, sl] = buf[b, sl] * spl

    def _scatter(dstv, buf):
        pltpu.sync_copy(buf, acc.at[dstv], add=True)

    _start_blk(0, dstv0, ewv0, rows0, sem0)

    @pl.loop(0, NB // 2)
    def _blocks(k):
        i = 2 * k
        _start_blk(i + 1, dstv1, ewv1, rows1, sem1)
        _wait_blk(i, ewv0, rows0, sem0)
        _scale(ewv0, rows0)
        _scatter(dstv0, rows0)
        _start_blk(i + 2, dstv0, ewv0, rows0, sem0)
        _wait_blk(i + 1, ewv1, rows1, sem1)
        _scale(ewv1, rows1)
        _scatter(dstv1, rows1)

    _wait_blk(NB - 1, ewv0, rows0, sem0)
    _scale(ewv0, rows0)
    _scatter(dstv0, rows0)

    plsc.subcore_barrier()
    for k in range(RPT // ZROWS):
        r0 = s * RPT + k * ZROWS
        pltpu.sync_copy(acc.at[pl.ds(r0, ZROWS)], out_hbm.at[c, pl.ds(r0, ZROWS)])


def _dense1_body(x_ref, w_ref, deg_ref, h1_ref, g1_ref):
    h1 = jnp.dot(x_ref[...], w_ref[...], preferred_element_type=jnp.float32)
    dinv = lax.rsqrt(deg_ref[...] + 1.0)
    h1_ref[...] = h1
    g1_ref[...] = h1 * dinv


def _dense1(x, W1, deg_col):
    return pl.pallas_call(
        _dense1_body,
        out_shape=(jax.ShapeDtypeStruct((N, D), jnp.float32),
                   jax.ShapeDtypeStruct((N, D), jnp.float32)),
    )(x, W1, deg_col)


def _dense2_body(s0_ref, s1_ref, h1_ref, deg_ref, gamma_ref, beta_ref, b1_ref,
                 w2_ref, h2_ref, g2_ref):
    dinv = lax.rsqrt(deg_ref[...] + 1.0)
    h = dinv * (s0_ref[...] + s1_ref[...]) + (dinv * dinv) * h1_ref[...] \
        + b1_ref[...]
    mean = jnp.mean(h, axis=0, keepdims=True)
    hm = h - mean
    var = jnp.mean(hm * hm, axis=0, keepdims=True)
    hn = hm * lax.rsqrt(var + EPS) * gamma_ref[...] + beta_ref[...]
    r = jnp.maximum(hn, 0.0)
    h2 = jnp.dot(r, w2_ref[...], preferred_element_type=jnp.float32)
    h2_ref[...] = h2
    g2_ref[...] = h2 * dinv


def _dense2(s0, s1, h1, deg_col, gamma, beta, b1, W2):
    return pl.pallas_call(
        _dense2_body,
        out_shape=(jax.ShapeDtypeStruct((N, D), jnp.float32),
                   jax.ShapeDtypeStruct((N, D), jnp.float32)),
    )(s0, s1, h1, deg_col, gamma, beta, b1, W2)


def _dense3_body(s0_ref, s1_ref, h2_ref, deg_ref, b2_ref, out_ref):
    dinv = lax.rsqrt(deg_ref[...] + 1.0)
    out_ref[...] = dinv * (s0_ref[...] + s1_ref[...]) \
        + (dinv * dinv) * h2_ref[...] + b2_ref[...]


def _dense3(s0, s1, h2, deg_col, b2):
    return pl.pallas_call(
        _dense3_body,
        out_shape=jax.ShapeDtypeStruct((N, D), jnp.float32),
    )(s0, s1, h2, deg_col, b2)


def kernel(t, x_nodes, edge_index, edge_weight, W1, b1, gamma1, beta1, W2, b2):
    src = edge_index[0]
    dst = edge_index[1]
    dst3 = dst.reshape(NW, NB, BLK)
    ew3 = edge_weight.reshape(NW, NB, BLK)

    degp = _deg_call(dst3, ew3)
    deg_col = (degp[0, :N] + degp[1, :N]).reshape(N, 1)

    h1, g1 = _dense1(x_nodes, W1, deg_col)
    s1 = _edge_call(g1, src, dst, edge_weight)
    h2, g2 = _dense2(s1[0, :N], s1[1, :N], h1, deg_col,
                     gamma1.reshape(1, D), beta1.reshape(1, D),
                     b1.reshape(1, D), W2)
    s2 = _edge_call(g2, src, dst, edge_weight)
    dz = _dense3(s2[0, :N], s2[1, :N], h2, deg_col, b2.reshape(1, D))
    return dz
